# baseline (speedup 1.0000x reference)
import functools

import jax
import jax.numpy as jnp
import numpy as np
from jax.experimental import pallas as pl
from jax.experimental.pallas import tpu as pltpu

_LAYER_CONFIGS = [
    (1, 10, 1),
    (10, 32, 2),
    (32, 64, 2),
    (64, 128, 2),
    (128, 256, 2),
    (256, 256, 2),
]
_BN_EPS = 1e-5

_BLOCK = 256
_BATCH_BLOCK = 1024

_SPATIAL = [(16, 16), (16, 16), (8, 8), (4, 4), (2, 2), (1, 1)]


def _fold_bn(w, b, g, be, rm, rv):
    scale = g / jnp.sqrt(rv + _BN_EPS)
    w_taps = jnp.transpose(w, (2, 3, 1, 0)).reshape(9, w.shape[1], w.shape[0])
    return w_taps * scale[None, None, :], (b - rm) * scale + be


def _layer1_stack(w_taps):
    hh = np.zeros((3, 16, 16), np.float32)
    vv = np.zeros((3, 16, 16), np.float32)
    for k in range(3):
        for j in range(16):
            if 0 <= j + k - 1 < 16:
                hh[k, j + k - 1, j] = 1.0
                vv[k, j, j + k - 1] = 1.0
    hh, vv = jnp.asarray(hh), jnp.asarray(vv)
    wt = w_taps[:, 0, :].reshape(3, 3, 10)
    wt = jnp.pad(wt, ((0, 0), (0, 0), (0, 6)))
    b = (wt[:, :, None, :, None] * hh[None, :, :, None, :]).sum(1)
    b = b.reshape(3, 16, _BLOCK)
    m = (vv[:, :, :, None, None] * b[:, None, None, :, :]).sum(0)
    return m.reshape(16 * _BLOCK, _BLOCK)


def _band_stack(w_taps, stride, w_in, w_out, khs):
    cin, cout = w_taps.shape[1], w_taps.shape[2]
    nr = len(khs)
    cw = cout * w_out
    a = np.zeros((9, nr, w_in, cw), np.float32)
    for rl, kh in enumerate(khs):
        for kw in range(3):
            t = kh * 3 + kw
            for jo in range(w_out):
                ji = stride * jo + kw - 1
                if 0 <= ji < w_in:
                    a[t, rl, ji, jo::w_out] = 1.0
    a = jnp.asarray(a)
    wq = jnp.repeat(w_taps, w_out, axis=2)
    m = (wq[:, None, :, None, :] * a[:, :, None, :, :]).sum(0)
    m = m.reshape(nr, cin * w_in, cw)
    m = jnp.pad(m, ((0, 0), (0, _BLOCK - cin * w_in), (0, _BLOCK - cw)))
    return m.reshape(nr * _BLOCK, _BLOCK)


def _bias_row(bias, w_out):
    row = jnp.repeat(bias, w_out)
    return jnp.pad(row, (0, _BLOCK - row.shape[0])).reshape(1, _BLOCK)


def _net_kernel(x_ref, w_ref, b_ref, o_ref, *, plan):
    x = x_ref[...].astype(jnp.bfloat16)

    blocks = []
    for r in range(16):
        acc = jnp.dot(x, w_ref[r * _BLOCK:(r + 1) * _BLOCK, :],
                      preferred_element_type=jnp.float32)
        y = jnp.maximum(acc + b_ref[0:1, :], 0.0)
        blocks.append(y.astype(jnp.bfloat16))
    h = jnp.concatenate(blocks, axis=1)

    n_layers = len(plan)
    for li, (w_off, bands) in enumerate(plan):
        outs = []
        for (nr, r0, m_off) in bands:
            seg = h[:, r0 * _BLOCK:(r0 + nr) * _BLOCK]
            mat = w_ref[w_off + m_off:w_off + m_off + nr * _BLOCK, :]
            acc = jnp.dot(seg, mat, preferred_element_type=jnp.float32)
            y = jnp.maximum(acc + b_ref[li + 1:li + 2, :], 0.0)
            if li < n_layers - 1:
                y = y.astype(jnp.bfloat16)
            outs.append(y)
        h = outs[0] if len(outs) == 1 else jnp.concatenate(outs, axis=1)
    o_ref[...] = h


def kernel(x, w0, b0, g0, be0, rm0, rv0, w1, b1, g1, be1, rm1, rv1,
           w2, b2, g2, be2, rm2, rv2, w3, b3, g3, be3, rm3, rv3,
           w4, b4, g4, be4, rm4, rv4, w5, b5, g5, be5, rm5, rv5):
    params = [
        (w0, b0, g0, be0, rm0, rv0),
        (w1, b1, g1, be1, rm1, rv1),
        (w2, b2, g2, be2, rm2, rv2),
        (w3, b3, g3, be3, rm3, rv3),
        (w4, b4, g4, be4, rm4, rv4),
        (w5, b5, g5, be5, rm5, rv5),
    ]
    n, cin0, h0, w0_ = x.shape
    assert cin0 == 1 and (h0, w0_) == (16, 16)

    wt1, bias1 = _fold_bn(*params[0])
    pieces = [_layer1_stack(wt1)]
    bias_rows = [_bias_row(bias1, 16)]

    plan = []
    w_off = 16 * _BLOCK
    for li in range(1, 6):
        _, _, stride = _LAYER_CONFIGS[li]
        h_in, w_in = _SPATIAL[li]
        h_out, w_out = (h_in + 1) // stride, (w_in + 1) // stride
        wt, bias = _fold_bn(*params[li])

        if h_in >= 3:
            stored_khs = (0, 1, 2)
        elif h_in == 2:
            stored_khs = (1, 2)
        else:
            stored_khs = (1,)
        pieces.append(_band_stack(wt, stride, w_in, w_out, stored_khs))
        bias_rows.append(_bias_row(bias, w_out))

        bands = []
        for k in range(h_out):
            rows = [r for r in (stride * k - 1, stride * k, stride * k + 1)
                    if 0 <= r < h_in]
            khs = tuple(r - (stride * k - 1) for r in rows)
            m_off = stored_khs.index(khs[0]) * _BLOCK
            bands.append((len(rows), rows[0], m_off))
        plan.append((w_off, tuple(bands)))
        w_off += len(stored_khs) * _BLOCK

    weights = jnp.concatenate(pieces, axis=0).astype(jnp.bfloat16)
    biases = jnp.concatenate(
        bias_rows + [jnp.zeros((8 - len(bias_rows), _BLOCK), jnp.float32)],
        axis=0).astype(jnp.float32)

    x_flat = x.reshape(n, 256)

    nb = _BATCH_BLOCK if n % _BATCH_BLOCK == 0 else 8
    out = pl.pallas_call(
        functools.partial(_net_kernel, plan=tuple(plan)),
        out_shape=jax.ShapeDtypeStruct((n, _BLOCK), jnp.float32),
        grid=(n // nb,),
        in_specs=[
            pl.BlockSpec((nb, 256), lambda i: (i, 0)),
            pl.BlockSpec(weights.shape, lambda i: (0, 0)),
            pl.BlockSpec(biases.shape, lambda i: (0, 0)),
        ],
        out_specs=pl.BlockSpec((nb, _BLOCK), lambda i: (i, 0)),
        compiler_params=pltpu.CompilerParams(
            dimension_semantics=("parallel",),
            vmem_limit_bytes=100 * 1024 * 1024,
        ),
    )(x_flat, weights, biases)
    return out.reshape(n, _BLOCK, 1, 1)


# speedup vs baseline: 1.0126x; 1.0126x over previous
import functools

import jax
import jax.numpy as jnp
import numpy as np
from jax.experimental import pallas as pl
from jax.experimental.pallas import tpu as pltpu

_LAYER_CONFIGS = [
    (1, 10, 1),
    (10, 32, 2),
    (32, 64, 2),
    (64, 128, 2),
    (128, 256, 2),
    (256, 256, 2),
]
_BN_EPS = 1e-5

_BLOCK = 256
_BATCH_BLOCK = 2048

_SPATIAL = [(16, 16), (16, 16), (8, 8), (4, 4), (2, 2), (1, 1)]


def _fold_bn(w, b, g, be, rm, rv):
    scale = g / jnp.sqrt(rv + _BN_EPS)
    w_taps = jnp.transpose(w, (2, 3, 1, 0)).reshape(9, w.shape[1], w.shape[0])
    return w_taps * scale[None, None, :], (b - rm) * scale + be


def _layer1_stack(w_taps):
    hh = np.zeros((3, 16, 16), np.float32)
    vv = np.zeros((3, 16, 16), np.float32)
    for k in range(3):
        for j in range(16):
            if 0 <= j + k - 1 < 16:
                hh[k, j + k - 1, j] = 1.0
                vv[k, j, j + k - 1] = 1.0
    hh, vv = jnp.asarray(hh), jnp.asarray(vv)
    wt = w_taps[:, 0, :].reshape(3, 3, 10)
    wt = jnp.pad(wt, ((0, 0), (0, 0), (0, 6)))
    b = (wt[:, :, None, :, None] * hh[None, :, :, None, :]).sum(1)
    b = b.reshape(3, 16, _BLOCK)
    m = (vv[:, :, :, None, None] * b[:, None, None, :, :]).sum(0)
    return m.reshape(16 * _BLOCK, _BLOCK)


def _band_stack(w_taps, stride, w_in, w_out, khs):
    cin, cout = w_taps.shape[1], w_taps.shape[2]
    nr = len(khs)
    cw = cout * w_out
    a = np.zeros((9, nr, w_in, cw), np.float32)
    for rl, kh in enumerate(khs):
        for kw in range(3):
            t = kh * 3 + kw
            for jo in range(w_out):
                ji = stride * jo + kw - 1
                if 0 <= ji < w_in:
                    a[t, rl, ji, jo::w_out] = 1.0
    a = jnp.asarray(a)
    wq = jnp.repeat(w_taps, w_out, axis=2)
    m = (wq[:, None, :, None, :] * a[:, :, None, :, :]).sum(0)
    m = m.reshape(nr, cin * w_in, cw)
    m = jnp.pad(m, ((0, 0), (0, _BLOCK - cin * w_in), (0, _BLOCK - cw)))
    return m.reshape(nr * _BLOCK, _BLOCK)


def _bias_row(bias, w_out):
    row = jnp.repeat(bias, w_out)
    return jnp.pad(row, (0, _BLOCK - row.shape[0])).reshape(1, _BLOCK)


def _net_kernel(x_ref, w_ref, b_ref, o_ref, *, plan):
    x = x_ref[...].astype(jnp.bfloat16)

    blocks = []
    for r in range(16):
        acc = jnp.dot(x, w_ref[r * _BLOCK:(r + 1) * _BLOCK, :],
                      preferred_element_type=jnp.float32)
        y = jnp.maximum(acc + b_ref[0:1, :], 0.0)
        blocks.append(y.astype(jnp.bfloat16))
    h = jnp.concatenate(blocks, axis=1)

    n_layers = len(plan)
    for li, (w_off, bands) in enumerate(plan):
        outs = []
        for (nr, r0, m_off) in bands:
            seg = h[:, r0 * _BLOCK:(r0 + nr) * _BLOCK]
            mat = w_ref[w_off + m_off:w_off + m_off + nr * _BLOCK, :]
            acc = jnp.dot(seg, mat, preferred_element_type=jnp.float32)
            y = jnp.maximum(acc + b_ref[li + 1:li + 2, :], 0.0)
            if li < n_layers - 1:
                y = y.astype(jnp.bfloat16)
            outs.append(y)
        h = outs[0] if len(outs) == 1 else jnp.concatenate(outs, axis=1)
    o_ref[...] = h


def kernel(x, w0, b0, g0, be0, rm0, rv0, w1, b1, g1, be1, rm1, rv1,
           w2, b2, g2, be2, rm2, rv2, w3, b3, g3, be3, rm3, rv3,
           w4, b4, g4, be4, rm4, rv4, w5, b5, g5, be5, rm5, rv5):
    params = [
        (w0, b0, g0, be0, rm0, rv0),
        (w1, b1, g1, be1, rm1, rv1),
        (w2, b2, g2, be2, rm2, rv2),
        (w3, b3, g3, be3, rm3, rv3),
        (w4, b4, g4, be4, rm4, rv4),
        (w5, b5, g5, be5, rm5, rv5),
    ]
    n, cin0, h0, w0_ = x.shape
    assert cin0 == 1 and (h0, w0_) == (16, 16)

    wt1, bias1 = _fold_bn(*params[0])
    pieces = [_layer1_stack(wt1)]
    bias_rows = [_bias_row(bias1, 16)]

    plan = []
    w_off = 16 * _BLOCK
    for li in range(1, 6):
        _, _, stride = _LAYER_CONFIGS[li]
        h_in, w_in = _SPATIAL[li]
        h_out, w_out = (h_in + 1) // stride, (w_in + 1) // stride
        wt, bias = _fold_bn(*params[li])

        if h_in >= 3:
            stored_khs = (0, 1, 2)
        elif h_in == 2:
            stored_khs = (1, 2)
        else:
            stored_khs = (1,)
        pieces.append(_band_stack(wt, stride, w_in, w_out, stored_khs))
        bias_rows.append(_bias_row(bias, w_out))

        bands = []
        for k in range(h_out):
            rows = [r for r in (stride * k - 1, stride * k, stride * k + 1)
                    if 0 <= r < h_in]
            khs = tuple(r - (stride * k - 1) for r in rows)
            m_off = stored_khs.index(khs[0]) * _BLOCK
            bands.append((len(rows), rows[0], m_off))
        plan.append((w_off, tuple(bands)))
        w_off += len(stored_khs) * _BLOCK

    weights = jnp.concatenate(pieces, axis=0).astype(jnp.bfloat16)
    biases = jnp.concatenate(
        bias_rows + [jnp.zeros((8 - len(bias_rows), _BLOCK), jnp.float32)],
        axis=0).astype(jnp.float32)

    x_flat = x.reshape(n, 256)

    nb = _BATCH_BLOCK if n % _BATCH_BLOCK == 0 else 8
    out = pl.pallas_call(
        functools.partial(_net_kernel, plan=tuple(plan)),
        out_shape=jax.ShapeDtypeStruct((n, _BLOCK), jnp.float32),
        grid=(n // nb,),
        in_specs=[
            pl.BlockSpec((nb, 256), lambda i: (i, 0)),
            pl.BlockSpec(weights.shape, lambda i: (0, 0)),
            pl.BlockSpec(biases.shape, lambda i: (0, 0)),
        ],
        out_specs=pl.BlockSpec((nb, _BLOCK), lambda i: (i, 0)),
        compiler_params=pltpu.CompilerParams(
            dimension_semantics=("parallel",),
            vmem_limit_bytes=100 * 1024 * 1024,
        ),
    )(x_flat, weights, biases)
    return out.reshape(n, _BLOCK, 1, 1)
